# baseline (speedup 1.0000x reference)
import functools

import numpy as np
import jax
import jax.numpy as jnp
from jax import lax
from jax.experimental import pallas as pl
from jax.experimental.pallas import tpu as pltpu
from jax.experimental.pallas import tpu_sc as plsc

_H = 512
_OUT = 384
_LO = 128
_SCALE = 1.25
_NTILES = 32
_RPT = _OUT // _NTILES
_JBLKS = _OUT // 16


def _bilinear_mat():
    inv = _H / (_H * _SCALE)
    o = np.arange(_LO, _LO + _OUT, dtype=np.float64)
    s = (o + 0.5) * inv - 0.5
    k = np.arange(_H, dtype=np.float64)
    w = np.maximum(0.0, 1.0 - np.abs(s[:, None] - k[None, :]))
    w = w / w.sum(1, keepdims=True)
    return w.astype(np.float32)


def _gather_maps():
    src = np.arange(_H)
    tgt = np.round(src.astype(np.float32) * np.float32(_SCALE)).astype(np.int64)
    r = tgt - _LO
    ok = (r >= 0) & (r < _OUT)
    idx = np.zeros((_OUT,), np.int32)
    hit = np.zeros((_OUT,), np.float32)
    idx[r[ok]] = src[ok]
    hit[r[ok]] = 1.0
    return idx, hit


_R = _bilinear_mat()
_IDX, _HIT = _gather_maps()
_RSRC = np.full((_NTILES, 16), _H, np.int32)
_ridx = _IDX.copy()
_ridx[_HIT == 0.0] = _H
_RSRC[:, :_RPT] = _ridx.reshape(_NTILES, _RPT)
_RSRC = _RSRC.reshape(-1)
_PLANE = _OUT * _OUT
_MROWS = _H + 8


def _tc_body(il_ref, ir_ref, r_ref, oil_ref, oir_ref):
    r = r_ref[...]
    rt = r_ref[...].T
    for x_ref, o_ref in ((il_ref, oil_ref), (ir_ref, oir_ref)):
        for p in range(3):
            t = jnp.dot(r, x_ref[p], preferred_element_type=jnp.float32)
            o_ref[p] = jnp.dot(t, rt, preferred_element_type=jnp.float32)


def _sc_flow(dl_hbm, dr_hbm, mfl_hbm, mfr_hbm, rsrc_hbm,
             csrc_hbm, chit_hbm,
             odl, odr, oml, omr,
             idx_v, idx1_v, csrc_v, chit_v,
             r0_v, r1_v, rm_v, o0_v, o1_v, om_v, sem):
    wid = lax.axis_index("s") * 2 + lax.axis_index("c")
    woff = pl.multiple_of(wid * 16, 16)
    pltpu.sync_copy(rsrc_hbm.at[pl.ds(woff, 16)], idx_v)
    pltpu.sync_copy(csrc_hbm, csrc_v)
    pltpu.sync_copy(chit_hbm, chit_v)
    idx1_v[...] = jnp.minimum(idx_v[...] + _H, 2 * _H - 1)

    for d_hbm, mf_hbm, od, omo in ((dl_hbm, mfl_hbm, odl, oml),
                                   (dr_hbm, mfr_hbm, odr, omr)):
        pltpu.async_copy(d_hbm.at[idx_v], r0_v, sem).wait()
        pltpu.async_copy(d_hbm.at[idx1_v], r1_v, sem).wait()
        pltpu.async_copy(mf_hbm.at[idx_v], rm_v, sem).wait()

        def jbody(j, carry):
            cidx = csrc_v[pl.ds(j * 16, 16)]
            chit = chit_v[pl.ds(j * 16, 16)]
            for r in range(_RPT):
                rsp = jnp.full((16,), r, jnp.int32)
                mg = plsc.load_gather(rm_v, [rsp, cidx]) * chit
                d0 = plsc.load_gather(r0_v, [rsp, cidx])
                d1 = plsc.load_gather(r1_v, [rsp, cidx])
                om_v[pl.ds(r * _OUT + j * 16, 16)] = mg
                sm = mg * jnp.float32(_SCALE)
                o0_v[pl.ds(r * _OUT + j * 16, 16)] = d0 * sm
                o1_v[pl.ds(r * _OUT + j * 16, 16)] = d1 * sm
            return carry

        lax.fori_loop(0, _JBLKS, jbody, 0)
        boff = pl.multiple_of(wid * (_RPT * _OUT), _RPT * _OUT)
        pltpu.sync_copy(o0_v, od.at[pl.ds(boff, _RPT * _OUT)])
        pltpu.sync_copy(o1_v, od.at[pl.ds(_PLANE + boff, _RPT * _OUT)])
        pltpu.sync_copy(om_v, omo.at[pl.ds(boff, _RPT * _OUT)])


_sc_call = functools.partial(
    pl.kernel,
    mesh=plsc.VectorSubcoreMesh(core_axis_name="c", subcore_axis_name="s"),
    compiler_params=pltpu.CompilerParams(
        use_tc_tiling_on_sc=False, needs_layout_passes=False),
    out_type=(
        jax.ShapeDtypeStruct((2 * _PLANE,), jnp.float32),
        jax.ShapeDtypeStruct((2 * _PLANE,), jnp.float32),
        jax.ShapeDtypeStruct((_PLANE,), jnp.float32),
        jax.ShapeDtypeStruct((_PLANE,), jnp.float32),
    ),
    scratch_types=[
        pltpu.VMEM((16,), jnp.int32),
        pltpu.VMEM((16,), jnp.int32),
        pltpu.VMEM((_OUT,), jnp.int32),
        pltpu.VMEM((_OUT,), jnp.float32),
        pltpu.VMEM((16, _H), jnp.float32),
        pltpu.VMEM((16, _H), jnp.float32),
        pltpu.VMEM((16, _H), jnp.float32),
        pltpu.VMEM((_RPT * _OUT,), jnp.float32),
        pltpu.VMEM((_RPT * _OUT,), jnp.float32),
        pltpu.VMEM((_RPT * _OUT,), jnp.float32),
        pltpu.SemaphoreType.DMA,
    ],
)(_sc_flow)


def kernel(img_left, img_right, dsp_left, dsp_right, mask_left, mask_right):
    oil, oir = pl.pallas_call(
        _tc_body,
        out_shape=(
            jax.ShapeDtypeStruct((3, _OUT, _OUT), jnp.float32),
            jax.ShapeDtypeStruct((3, _OUT, _OUT), jnp.float32),
        ),
    )(img_left, img_right, jnp.asarray(_R))

    mfl = jnp.pad(mask_left.astype(jnp.float32), ((0, _MROWS - _H), (0, 0)))
    mfr = jnp.pad(mask_right.astype(jnp.float32), ((0, _MROWS - _H), (0, 0)))
    odl, odr, oml, omr = _sc_call(
        dsp_left.reshape(2 * _H, _H),
        dsp_right.reshape(2 * _H, _H),
        mfl, mfr,
        jnp.asarray(_RSRC),
        jnp.asarray(_IDX), jnp.asarray(_HIT),
    )

    return (oil, oir,
            odl.reshape(2, _OUT, _OUT), odr.reshape(2, _OUT, _OUT),
            oml.reshape(_OUT, _OUT), omr.reshape(_OUT, _OUT))


# speedup vs baseline: 1.0112x; 1.0112x over previous
import functools

import numpy as np
import jax
import jax.numpy as jnp
from jax import lax
from jax.experimental import pallas as pl
from jax.experimental.pallas import tpu as pltpu
from jax.experimental.pallas import tpu_sc as plsc

_H = 512
_OUT = 384
_LO = 128
_SCALE = 1.25
_NTILES = 32
_RPT = _OUT // _NTILES
_JBLKS = _OUT // 16


def _bilinear_mat():
    inv = _H / (_H * _SCALE)
    o = np.arange(_LO, _LO + _OUT, dtype=np.float64)
    s = (o + 0.5) * inv - 0.5
    k = np.arange(_H, dtype=np.float64)
    w = np.maximum(0.0, 1.0 - np.abs(s[:, None] - k[None, :]))
    w = w / w.sum(1, keepdims=True)
    return w.astype(np.float32)


def _gather_maps():
    src = np.arange(_H)
    tgt = np.round(src.astype(np.float32) * np.float32(_SCALE)).astype(np.int64)
    r = tgt - _LO
    ok = (r >= 0) & (r < _OUT)
    idx = np.zeros((_OUT,), np.int32)
    hit = np.zeros((_OUT,), np.float32)
    idx[r[ok]] = src[ok]
    hit[r[ok]] = 1.0
    return idx, hit


_R = _bilinear_mat()
_IDX, _HIT = _gather_maps()
_RSRC = np.full((_NTILES, 16), _H, np.int32)
_ridx = _IDX.copy()
_ridx[_HIT == 0.0] = _H
_RSRC[:, :_RPT] = _ridx.reshape(_NTILES, _RPT)
_RSRC = _RSRC.reshape(-1)
_PLANE = _OUT * _OUT
_MROWS = _H + 8


def _tc_body(il_ref, ir_ref, r_ref, oil_ref, oir_ref):
    r = r_ref[...]
    rt = r_ref[...].T
    for x_ref, o_ref in ((il_ref, oil_ref), (ir_ref, oir_ref)):
        for p in range(3):
            t = jnp.dot(r, x_ref[p], preferred_element_type=jnp.float32)
            o_ref[p] = jnp.dot(t, rt, preferred_element_type=jnp.float32)


def _sc_flow(dl_hbm, dr_hbm, mfl_hbm, mfr_hbm, rsrc_hbm,
             csrc_hbm, chit_hbm,
             odl, odr, oml, omr,
             idx_v, idx1_v, csrc_v, chit_v,
             r0_v, r1_v, rm_v, o0_v, o1_v, om_v, sem):
    wid = lax.axis_index("s") * 2 + lax.axis_index("c")
    woff = pl.multiple_of(wid * 16, 16)
    pltpu.sync_copy(rsrc_hbm.at[pl.ds(woff, 16)], idx_v)
    pltpu.sync_copy(csrc_hbm, csrc_v)
    pltpu.sync_copy(chit_hbm, chit_v)
    idx1_v[...] = jnp.minimum(idx_v[...] + _H, 2 * _H - 1)

    for d_hbm, mf_hbm, od, omo in ((dl_hbm, mfl_hbm, odl, oml),
                                   (dr_hbm, mfr_hbm, odr, omr)):
        pltpu.async_copy(d_hbm.at[idx_v], r0_v, sem).wait()
        pltpu.async_copy(d_hbm.at[idx1_v], r1_v, sem).wait()
        pltpu.async_copy(mf_hbm.at[idx_v], rm_v, sem).wait()

        @plsc.parallel_loop(0, _JBLKS, unroll=2)
        def jbody(j):
            cidx = csrc_v[pl.ds(j * 16, 16)]
            chit = chit_v[pl.ds(j * 16, 16)]
            for r in range(_RPT):
                rsp = jnp.full((16,), r, jnp.int32)
                mg = plsc.load_gather(rm_v, [rsp, cidx]) * chit
                d0 = plsc.load_gather(r0_v, [rsp, cidx])
                d1 = plsc.load_gather(r1_v, [rsp, cidx])
                om_v[pl.ds(r * _OUT + j * 16, 16)] = mg
                sm = mg * jnp.float32(_SCALE)
                o0_v[pl.ds(r * _OUT + j * 16, 16)] = d0 * sm
                o1_v[pl.ds(r * _OUT + j * 16, 16)] = d1 * sm
        boff = pl.multiple_of(wid * (_RPT * _OUT), _RPT * _OUT)
        pltpu.sync_copy(o0_v, od.at[pl.ds(boff, _RPT * _OUT)])
        pltpu.sync_copy(o1_v, od.at[pl.ds(_PLANE + boff, _RPT * _OUT)])
        pltpu.sync_copy(om_v, omo.at[pl.ds(boff, _RPT * _OUT)])


_sc_call = functools.partial(
    pl.kernel,
    mesh=plsc.VectorSubcoreMesh(core_axis_name="c", subcore_axis_name="s"),
    compiler_params=pltpu.CompilerParams(
        use_tc_tiling_on_sc=False, needs_layout_passes=False),
    out_type=(
        jax.ShapeDtypeStruct((2 * _PLANE,), jnp.float32),
        jax.ShapeDtypeStruct((2 * _PLANE,), jnp.float32),
        jax.ShapeDtypeStruct((_PLANE,), jnp.float32),
        jax.ShapeDtypeStruct((_PLANE,), jnp.float32),
    ),
    scratch_types=[
        pltpu.VMEM((16,), jnp.int32),
        pltpu.VMEM((16,), jnp.int32),
        pltpu.VMEM((_OUT,), jnp.int32),
        pltpu.VMEM((_OUT,), jnp.float32),
        pltpu.VMEM((16, _H), jnp.float32),
        pltpu.VMEM((16, _H), jnp.float32),
        pltpu.VMEM((16, _H), jnp.float32),
        pltpu.VMEM((_RPT * _OUT,), jnp.float32),
        pltpu.VMEM((_RPT * _OUT,), jnp.float32),
        pltpu.VMEM((_RPT * _OUT,), jnp.float32),
        pltpu.SemaphoreType.DMA,
    ],
)(_sc_flow)


def kernel(img_left, img_right, dsp_left, dsp_right, mask_left, mask_right):
    oil, oir = pl.pallas_call(
        _tc_body,
        out_shape=(
            jax.ShapeDtypeStruct((3, _OUT, _OUT), jnp.float32),
            jax.ShapeDtypeStruct((3, _OUT, _OUT), jnp.float32),
        ),
    )(img_left, img_right, jnp.asarray(_R))

    mfl = jnp.pad(mask_left.astype(jnp.float32), ((0, _MROWS - _H), (0, 0)))
    mfr = jnp.pad(mask_right.astype(jnp.float32), ((0, _MROWS - _H), (0, 0)))
    odl, odr, oml, omr = _sc_call(
        dsp_left.reshape(2 * _H, _H),
        dsp_right.reshape(2 * _H, _H),
        mfl, mfr,
        jnp.asarray(_RSRC),
        jnp.asarray(_IDX), jnp.asarray(_HIT),
    )

    return (oil, oir,
            odl.reshape(2, _OUT, _OUT), odr.reshape(2, _OUT, _OUT),
            oml.reshape(_OUT, _OUT), omr.reshape(_OUT, _OUT))


# speedup vs baseline: 1.0141x; 1.0029x over previous
import functools

import numpy as np
import jax
import jax.numpy as jnp
from jax import lax
from jax.experimental import pallas as pl
from jax.experimental.pallas import tpu as pltpu
from jax.experimental.pallas import tpu_sc as plsc

_H = 512
_OUT = 384
_LO = 128
_SCALE = 1.25
_NTILES = 32
_RPT = _OUT // _NTILES
_JBLKS = _OUT // 16


def _bilinear_mat():
    inv = _H / (_H * _SCALE)
    o = np.arange(_LO, _LO + _OUT, dtype=np.float64)
    s = (o + 0.5) * inv - 0.5
    k = np.arange(_H, dtype=np.float64)
    w = np.maximum(0.0, 1.0 - np.abs(s[:, None] - k[None, :]))
    w = w / w.sum(1, keepdims=True)
    return w.astype(np.float32)


def _gather_maps():
    src = np.arange(_H)
    tgt = np.round(src.astype(np.float32) * np.float32(_SCALE)).astype(np.int64)
    r = tgt - _LO
    ok = (r >= 0) & (r < _OUT)
    idx = np.zeros((_OUT,), np.int32)
    hit = np.zeros((_OUT,), np.float32)
    idx[r[ok]] = src[ok]
    hit[r[ok]] = 1.0
    return idx, hit


_R = _bilinear_mat()
_IDX, _HIT = _gather_maps()
_RSRC = np.full((_NTILES, 16), _H, np.int32)
_ridx = _IDX.copy()
_ridx[_HIT == 0.0] = _H
_RSRC[:, :_RPT] = _ridx.reshape(_NTILES, _RPT)
_RSRC = _RSRC.reshape(-1)
_PLANE = _OUT * _OUT
_MROWS = _H + 8


def _tc_body(il_ref, ir_ref, r_ref, oil_ref, oir_ref):
    r = r_ref[...]
    rt = r_ref[...].T
    for x_ref, o_ref in ((il_ref, oil_ref), (ir_ref, oir_ref)):
        for p in range(3):
            t = jnp.dot(r, x_ref[p], preferred_element_type=jnp.float32)
            o_ref[p] = jnp.dot(t, rt, preferred_element_type=jnp.float32)


def _sc_flow(dl_hbm, dr_hbm, mfl_hbm, mfr_hbm, rsrc_hbm,
             csrc_hbm, chit_hbm,
             odl, odr, oml, omr,
             idx_v, idx1_v, csrc_v, chit_v,
             r0_v, r1_v, rm_v, o0_v, o1_v, om_v, sem):
    wid = lax.axis_index("s") * 2 + lax.axis_index("c")
    woff = pl.multiple_of(wid * 16, 16)
    pltpu.sync_copy(rsrc_hbm.at[pl.ds(woff, 16)], idx_v)
    pltpu.sync_copy(csrc_hbm, csrc_v)
    pltpu.sync_copy(chit_hbm, chit_v)
    idx1_v[...] = jnp.minimum(idx_v[...] + _H, 2 * _H - 1)

    for d_hbm, mf_hbm, od, omo in ((dl_hbm, mfl_hbm, odl, oml),
                                   (dr_hbm, mfr_hbm, odr, omr)):
        with jax.named_scope("rowgather"):
            pltpu.async_copy(d_hbm.at[idx_v], r0_v, sem).wait()
            pltpu.async_copy(d_hbm.at[idx1_v], r1_v, sem).wait()
            pltpu.async_copy(mf_hbm.at[idx_v], rm_v, sem).wait()

        with jax.named_scope("colgather"):
            @plsc.parallel_loop(0, _JBLKS, unroll=2)
            def jbody(j):
                cidx = csrc_v[pl.ds(j * 16, 16)]
                chit = chit_v[pl.ds(j * 16, 16)]
                for r in range(_RPT):
                    rsp = jnp.full((16,), r, jnp.int32)
                    mg = plsc.load_gather(rm_v, [rsp, cidx]) * chit
                    d0 = plsc.load_gather(r0_v, [rsp, cidx])
                    d1 = plsc.load_gather(r1_v, [rsp, cidx])
                    om_v[pl.ds(r * _OUT + j * 16, 16)] = mg
                    sm = mg * jnp.float32(_SCALE)
                    o0_v[pl.ds(r * _OUT + j * 16, 16)] = d0 * sm
                    o1_v[pl.ds(r * _OUT + j * 16, 16)] = d1 * sm
        with jax.named_scope("writeout"):
            boff = pl.multiple_of(wid * (_RPT * _OUT), _RPT * _OUT)
            pltpu.sync_copy(o0_v, od.at[pl.ds(boff, _RPT * _OUT)])
            pltpu.sync_copy(o1_v, od.at[pl.ds(_PLANE + boff, _RPT * _OUT)])
            pltpu.sync_copy(om_v, omo.at[pl.ds(boff, _RPT * _OUT)])


_sc_call = functools.partial(
    pl.kernel,
    mesh=plsc.VectorSubcoreMesh(core_axis_name="c", subcore_axis_name="s"),
    compiler_params=pltpu.CompilerParams(
        use_tc_tiling_on_sc=False, needs_layout_passes=False),
    out_type=(
        jax.ShapeDtypeStruct((2 * _PLANE,), jnp.float32),
        jax.ShapeDtypeStruct((2 * _PLANE,), jnp.float32),
        jax.ShapeDtypeStruct((_PLANE,), jnp.float32),
        jax.ShapeDtypeStruct((_PLANE,), jnp.float32),
    ),
    scratch_types=[
        pltpu.VMEM((16,), jnp.int32),
        pltpu.VMEM((16,), jnp.int32),
        pltpu.VMEM((_OUT,), jnp.int32),
        pltpu.VMEM((_OUT,), jnp.float32),
        pltpu.VMEM((16, _H), jnp.float32),
        pltpu.VMEM((16, _H), jnp.float32),
        pltpu.VMEM((16, _H), jnp.float32),
        pltpu.VMEM((_RPT * _OUT,), jnp.float32),
        pltpu.VMEM((_RPT * _OUT,), jnp.float32),
        pltpu.VMEM((_RPT * _OUT,), jnp.float32),
        pltpu.SemaphoreType.DMA,
    ],
)(_sc_flow)


def kernel(img_left, img_right, dsp_left, dsp_right, mask_left, mask_right):
    oil, oir = pl.pallas_call(
        _tc_body,
        out_shape=(
            jax.ShapeDtypeStruct((3, _OUT, _OUT), jnp.float32),
            jax.ShapeDtypeStruct((3, _OUT, _OUT), jnp.float32),
        ),
    )(img_left, img_right, jnp.asarray(_R))

    mfl = jnp.pad(mask_left.astype(jnp.float32), ((0, _MROWS - _H), (0, 0)))
    mfr = jnp.pad(mask_right.astype(jnp.float32), ((0, _MROWS - _H), (0, 0)))
    odl, odr, oml, omr = _sc_call(
        dsp_left.reshape(2 * _H, _H),
        dsp_right.reshape(2 * _H, _H),
        mfl, mfr,
        jnp.asarray(_RSRC),
        jnp.asarray(_IDX), jnp.asarray(_HIT),
    )

    return (oil, oir,
            odl.reshape(2, _OUT, _OUT), odr.reshape(2, _OUT, _OUT),
            oml.reshape(_OUT, _OUT), omr.reshape(_OUT, _OUT))


# speedup vs baseline: 1.6865x; 1.6631x over previous
import functools

import numpy as np
import jax
import jax.numpy as jnp
from jax import lax
from jax.experimental import pallas as pl
from jax.experimental.pallas import tpu as pltpu
from jax.experimental.pallas import tpu_sc as plsc

_H = 512
_OUT = 384
_LO = 128
_SCALE = 1.25
_NTILES = 32
_RPT = _OUT // _NTILES
_JBLKS = _OUT // 16
_BROWS = 16
_ZROW = _BROWS - 1


def _bilinear_mat():
    inv = _H / (_H * _SCALE)
    o = np.arange(_LO, _LO + _OUT, dtype=np.float64)
    s = (o + 0.5) * inv - 0.5
    k = np.arange(_H, dtype=np.float64)
    w = np.maximum(0.0, 1.0 - np.abs(s[:, None] - k[None, :]))
    w = w / w.sum(1, keepdims=True)
    return w.astype(np.float32)


def _gather_maps():
    src = np.arange(_H)
    tgt = np.round(src.astype(np.float32) * np.float32(_SCALE)).astype(np.int64)
    r = tgt - _LO
    ok = (r >= 0) & (r < _OUT)
    idx = np.full((_OUT,), -1, np.int64)
    idx[r[ok]] = src[ok]
    return idx


def _rlo(w):
    return max(0, (4 * (_LO + _RPT * w)) // 5 - 1)


def _flat_index_table():
    idx = _gather_maps()
    tab = np.zeros((_NTILES, _RPT * _OUT), np.int32)
    for w in range(_NTILES):
        rlo = _rlo(w)
        for r in range(_RPT):
            sy = idx[w * _RPT + r]
            for c in range(_OUT):
                sx = idx[c]
                if sy >= 0 and sx >= 0:
                    lrow = sy - rlo
                    assert 0 <= lrow < _RPT, (w, r, sy, rlo)
                    tab[w, r * _OUT + c] = lrow * _H + sx
                else:
                    tab[w, r * _OUT + c] = _ZROW * _H
        assert _rlo(w) + _RPT <= _H
    return tab.reshape(-1)


_R = _bilinear_mat()
_CFLAT = _flat_index_table()
_PLANE = _OUT * _OUT
_TBLK = _RPT * _OUT


def _tc_body(il_ref, ir_ref, r_ref, oil_ref, oir_ref):
    r = r_ref[...]
    rt = r_ref[...].T
    for x_ref, o_ref in ((il_ref, oil_ref), (ir_ref, oir_ref)):
        for p in range(3):
            t = jnp.dot(r, x_ref[p], preferred_element_type=jnp.float32)
            o_ref[p] = jnp.dot(t, rt, preferred_element_type=jnp.float32)


def _sc_flow(dl_hbm, dr_hbm, mfl_hbm, mfr_hbm, cflat_hbm,
             odl, odr, oml, omr,
             cidx_v, b0_v, b1_v, bm_v, c0_v, c1_v, cm_v,
             o0_v, o1_v, om_v, sem0, sem1):
    wid = lax.axis_index("s") * 2 + lax.axis_index("c")
    rlo = (4 * (_LO + _RPT * wid)) // 5 - 1
    coff = pl.multiple_of(wid * _TBLK, _TBLK)
    pltpu.sync_copy(cflat_hbm.at[pl.ds(coff, _TBLK)], cidx_v)

    zero16 = jnp.zeros((16,), jnp.float32)
    for buf in (b0_v, b1_v, bm_v, c0_v, c1_v, cm_v):
        for k in range(_H // 16):
            buf[_ZROW, pl.ds(k * 16, 16)] = zero16

    cpy = []
    for d_hbm, mf_hbm, bufs, sem in ((dl_hbm, mfl_hbm, (b0_v, b1_v, bm_v), sem0),
                                     (dr_hbm, mfr_hbm, (c0_v, c1_v, cm_v), sem1)):
        b0, b1, bm = bufs
        cpy.append((
            pltpu.make_async_copy(d_hbm.at[0, pl.ds(rlo, _RPT)],
                                  b0.at[pl.ds(0, _RPT)], sem),
            pltpu.make_async_copy(d_hbm.at[1, pl.ds(rlo, _RPT)],
                                  b1.at[pl.ds(0, _RPT)], sem),
            pltpu.make_async_copy(mf_hbm.at[pl.ds(rlo, _RPT)],
                                  bm.at[pl.ds(0, _RPT)], sem),
        ))
    for side in cpy:
        for c in side:
            c.start()

    boff = pl.multiple_of(wid * _RPT, _RPT)
    for (b0, b1, bm), copies, od, omo in (
            ((b0_v, b1_v, bm_v), cpy[0], odl, oml),
            ((c0_v, c1_v, cm_v), cpy[1], odr, omr)):
        with jax.named_scope("drain"):
            for c in copies:
                c.wait()

        with jax.named_scope("colgather"):
            @plsc.parallel_loop(0, _JBLKS, unroll=2)
            def jbody(j):
                for r in range(_RPT):
                    idx16 = cidx_v[pl.ds(r * _OUT + j * 16, 16)]
                    ridx = jax.lax.shift_right_logical(idx16, 9)
                    kidx = jax.lax.bitwise_and(idx16, jnp.int32(_H - 1))
                    mg = plsc.load_gather(bm, [ridx, kidx])
                    d0 = plsc.load_gather(b0, [ridx, kidx])
                    d1 = plsc.load_gather(b1, [ridx, kidx])
                    om_v[r, pl.ds(j * 16, 16)] = mg
                    sm = mg * jnp.float32(_SCALE)
                    o0_v[r, pl.ds(j * 16, 16)] = d0 * sm
                    o1_v[r, pl.ds(j * 16, 16)] = d1 * sm

        with jax.named_scope("writeout"):
            pltpu.sync_copy(o0_v, od.at[0, pl.ds(boff, _RPT)])
            pltpu.sync_copy(o1_v, od.at[1, pl.ds(boff, _RPT)])
            pltpu.sync_copy(om_v, omo.at[pl.ds(boff, _RPT)])


_sc_call = functools.partial(
    pl.kernel,
    mesh=plsc.VectorSubcoreMesh(core_axis_name="c", subcore_axis_name="s"),
    compiler_params=pltpu.CompilerParams(
        use_tc_tiling_on_sc=False, needs_layout_passes=False),
    out_type=(
        jax.ShapeDtypeStruct((2, _OUT, _OUT), jnp.float32),
        jax.ShapeDtypeStruct((2, _OUT, _OUT), jnp.float32),
        jax.ShapeDtypeStruct((_OUT, _OUT), jnp.float32),
        jax.ShapeDtypeStruct((_OUT, _OUT), jnp.float32),
    ),
    scratch_types=[
        pltpu.VMEM((_TBLK,), jnp.int32),
        pltpu.VMEM((_BROWS, _H), jnp.float32),
        pltpu.VMEM((_BROWS, _H), jnp.float32),
        pltpu.VMEM((_BROWS, _H), jnp.float32),
        pltpu.VMEM((_BROWS, _H), jnp.float32),
        pltpu.VMEM((_BROWS, _H), jnp.float32),
        pltpu.VMEM((_BROWS, _H), jnp.float32),
        pltpu.VMEM((_RPT, _OUT), jnp.float32),
        pltpu.VMEM((_RPT, _OUT), jnp.float32),
        pltpu.VMEM((_RPT, _OUT), jnp.float32),
        pltpu.SemaphoreType.DMA,
        pltpu.SemaphoreType.DMA,
    ],
)(_sc_flow)


def kernel(img_left, img_right, dsp_left, dsp_right, mask_left, mask_right):
    oil, oir = pl.pallas_call(
        _tc_body,
        out_shape=(
            jax.ShapeDtypeStruct((3, _OUT, _OUT), jnp.float32),
            jax.ShapeDtypeStruct((3, _OUT, _OUT), jnp.float32),
        ),
    )(img_left, img_right, jnp.asarray(_R))

    odl, odr, oml, omr = _sc_call(
        dsp_left, dsp_right,
        mask_left.astype(jnp.float32),
        mask_right.astype(jnp.float32),
        jnp.asarray(_CFLAT),
    )

    return (oil, oir, odl, odr, oml, omr)


# speedup vs baseline: 1.7444x; 1.0343x over previous
import functools

import numpy as np
import jax
import jax.numpy as jnp
from jax import lax
from jax.experimental import pallas as pl
from jax.experimental.pallas import tpu as pltpu
from jax.experimental.pallas import tpu_sc as plsc

_H = 512
_OUT = 384
_LO = 128
_SCALE = 1.25
_NTILES = 32
_RPT = _OUT // _NTILES
_BROWS = 16
_SIDE = _BROWS * _H
_TBLK = _RPT * _OUT
_PLANE = _OUT * _OUT
_NPOS = 2 * _TBLK // 16


def _bilinear_mat():
    inv = _H / (_H * _SCALE)
    o = np.arange(_LO, _LO + _OUT, dtype=np.float64)
    s = (o + 0.5) * inv - 0.5
    k = np.arange(_H, dtype=np.float64)
    w = np.maximum(0.0, 1.0 - np.abs(s[:, None] - k[None, :]))
    w = w / w.sum(1, keepdims=True)
    return w.astype(np.float32)


def _gather_maps():
    src = np.arange(_H)
    tgt = np.round(src.astype(np.float32) * np.float32(_SCALE)).astype(np.int64)
    r = tgt - _LO
    ok = (r >= 0) & (r < _OUT)
    idx = np.full((_OUT,), -1, np.int64)
    idx[r[ok]] = src[ok]
    return idx


def _rlo(w):
    return (4 * (_LO + _RPT * w)) // 5 - 1


def _flat_index_table():
    idx = _gather_maps()
    tab = np.zeros((_NTILES, 2 * _TBLK), np.int32)
    for w in range(_NTILES):
        rlo = _rlo(w)
        assert 0 <= rlo and rlo + _RPT <= _H
        for s in range(2):
            base = s * _SIDE
            gate = base + (_BROWS - 1) * _H
            for r in range(_RPT):
                sy = idx[w * _RPT + r]
                for c in range(_OUT):
                    sx = idx[c]
                    p = s * _TBLK + r * _OUT + c
                    if sy >= 0 and sx >= 0:
                        lrow = sy - rlo
                        assert 0 <= lrow < _RPT, (w, r, sy, rlo)
                        tab[w, p] = base + lrow * _H + sx
                    else:
                        tab[w, p] = gate
    return tab.reshape(-1)


_R = _bilinear_mat()
_CFLAT = _flat_index_table()


def _tc_body(il_ref, ir_ref, r_ref, oil_ref, oir_ref):
    r = r_ref[...]
    rt = r_ref[...].T
    for x_ref, o_ref in ((il_ref, oil_ref), (ir_ref, oir_ref)):
        for p in range(3):
            t = jnp.dot(r, x_ref[p], preferred_element_type=jnp.float32)
            o_ref[p] = jnp.dot(t, rt, preferred_element_type=jnp.float32)


def _sc_flow(dl_hbm, dr_hbm, mfl_hbm, mfr_hbm, cflat_hbm,
             odl, odr, oml, omr,
             cidx_v, b0_v, b1_v, bm_v, o0_v, o1_v, om_v, sem):
    wid = lax.axis_index("s") * 2 + lax.axis_index("c")
    rlo = (4 * (_LO + _RPT * wid)) // 5 - 1
    boff = pl.multiple_of(rlo * _H, 8)
    coff = pl.multiple_of(wid * (2 * _TBLK), 2 * _TBLK)
    idx_cp = pltpu.make_async_copy(
        cflat_hbm.at[pl.ds(coff, 2 * _TBLK)], cidx_v, sem)
    idx_cp.start()

    zero16 = jnp.zeros((16,), jnp.float32)
    gate0 = (_BROWS - 1) * _H

    @plsc.parallel_loop(0, _H // 16, unroll=2)
    def zbody(k):
        for buf in (b0_v, b1_v, bm_v):
            buf[pl.ds(gate0 + k * 16, 16)] = zero16
            buf[pl.ds(_SIDE + gate0 + k * 16, 16)] = zero16

    n = _RPT * _H
    copies = []
    for s, (d_hbm, mf_hbm) in enumerate(((dl_hbm, mfl_hbm), (dr_hbm, mfr_hbm))):
        sb = s * _SIDE
        copies += [
            pltpu.make_async_copy(d_hbm.at[pl.ds(boff, n)],
                                  b0_v.at[pl.ds(sb, n)], sem),
            pltpu.make_async_copy(d_hbm.at[pl.ds(_H * _H + boff, n)],
                                  b1_v.at[pl.ds(sb, n)], sem),
            pltpu.make_async_copy(mf_hbm.at[pl.ds(boff, n)],
                                  bm_v.at[pl.ds(sb, n)], sem),
        ]
    for c in copies:
        c.start()
    with jax.named_scope("drain"):
        idx_cp.wait()
        for c in copies:
            c.wait()

    with jax.named_scope("colgather"):
        @plsc.parallel_loop(0, _NPOS, unroll=2)
        def jbody(i):
            pos = i * 16
            idx16 = cidx_v[pl.ds(pos, 16)]
            mg = plsc.load_gather(bm_v, [idx16])
            d0 = plsc.load_gather(b0_v, [idx16])
            d1 = plsc.load_gather(b1_v, [idx16])
            om_v[pl.ds(pos, 16)] = mg
            sm = mg * jnp.float32(_SCALE)
            o0_v[pl.ds(pos, 16)] = d0 * sm
            o1_v[pl.ds(pos, 16)] = d1 * sm

    with jax.named_scope("writeout"):
        ooff = pl.multiple_of(wid * _TBLK, _TBLK)
        pltpu.sync_copy(o0_v.at[pl.ds(0, _TBLK)], odl.at[pl.ds(ooff, _TBLK)])
        pltpu.sync_copy(o1_v.at[pl.ds(0, _TBLK)],
                        odl.at[pl.ds(_PLANE + ooff, _TBLK)])
        pltpu.sync_copy(om_v.at[pl.ds(0, _TBLK)], oml.at[pl.ds(ooff, _TBLK)])
        pltpu.sync_copy(o0_v.at[pl.ds(_TBLK, _TBLK)],
                        odr.at[pl.ds(ooff, _TBLK)])
        pltpu.sync_copy(o1_v.at[pl.ds(_TBLK, _TBLK)],
                        odr.at[pl.ds(_PLANE + ooff, _TBLK)])
        pltpu.sync_copy(om_v.at[pl.ds(_TBLK, _TBLK)], omr.at[pl.ds(ooff, _TBLK)])


_sc_call = functools.partial(
    pl.kernel,
    mesh=plsc.VectorSubcoreMesh(core_axis_name="c", subcore_axis_name="s"),
    compiler_params=pltpu.CompilerParams(
        use_tc_tiling_on_sc=False, needs_layout_passes=False),
    out_type=(
        jax.ShapeDtypeStruct((2 * _PLANE,), jnp.float32),
        jax.ShapeDtypeStruct((2 * _PLANE,), jnp.float32),
        jax.ShapeDtypeStruct((_PLANE,), jnp.float32),
        jax.ShapeDtypeStruct((_PLANE,), jnp.float32),
    ),
    scratch_types=[
        pltpu.VMEM((2 * _TBLK,), jnp.int32),
        pltpu.VMEM((2 * _SIDE,), jnp.float32),
        pltpu.VMEM((2 * _SIDE,), jnp.float32),
        pltpu.VMEM((2 * _SIDE,), jnp.float32),
        pltpu.VMEM((2 * _TBLK,), jnp.float32),
        pltpu.VMEM((2 * _TBLK,), jnp.float32),
        pltpu.VMEM((2 * _TBLK,), jnp.float32),
        pltpu.SemaphoreType.DMA,
    ],
)(_sc_flow)


def kernel(img_left, img_right, dsp_left, dsp_right, mask_left, mask_right):
    oil, oir = pl.pallas_call(
        _tc_body,
        out_shape=(
            jax.ShapeDtypeStruct((3, _OUT, _OUT), jnp.float32),
            jax.ShapeDtypeStruct((3, _OUT, _OUT), jnp.float32),
        ),
    )(img_left, img_right, jnp.asarray(_R))

    odl, odr, oml, omr = _sc_call(
        dsp_left.reshape(-1), dsp_right.reshape(-1),
        mask_left.astype(jnp.float32).reshape(-1),
        mask_right.astype(jnp.float32).reshape(-1),
        jnp.asarray(_CFLAT),
    )

    return (oil, oir,
            odl.reshape(2, _OUT, _OUT), odr.reshape(2, _OUT, _OUT),
            oml.reshape(_OUT, _OUT), omr.reshape(_OUT, _OUT))


# speedup vs baseline: 1.7573x; 1.0074x over previous
import functools

import numpy as np
import jax
import jax.numpy as jnp
from jax import lax
from jax.experimental import pallas as pl
from jax.experimental.pallas import tpu as pltpu
from jax.experimental.pallas import tpu_sc as plsc

_H = 512
_OUT = 384
_W0 = 96
_WROWS = 320
_LO = 128
_SCALE = 1.25
_NTILES = 32
_RPT = _OUT // _NTILES
_BROWS = 16
_SIDE = _BROWS * _H
_TBLK = _RPT * _OUT
_PLANE = _OUT * _OUT
_NPOS = 2 * _TBLK // 16


def _bilinear_mat():
    inv = _H / (_H * _SCALE)
    o = np.arange(_LO, _LO + _OUT, dtype=np.float64)
    s = (o + 0.5) * inv - 0.5
    k = np.arange(_H, dtype=np.float64)
    w = np.maximum(0.0, 1.0 - np.abs(s[:, None] - k[None, :]))
    w = w / w.sum(1, keepdims=True)
    return w.astype(np.float32)


def _gather_maps():
    src = np.arange(_H)
    tgt = np.round(src.astype(np.float32) * np.float32(_SCALE)).astype(np.int64)
    r = tgt - _LO
    ok = (r >= 0) & (r < _OUT)
    idx = np.full((_OUT,), -1, np.int64)
    idx[r[ok]] = src[ok]
    return idx


def _rlo(w):
    return (4 * (_LO + _RPT * w)) // 5 - 1


def _flat_index_table():
    idx = _gather_maps()
    tab = np.zeros((_NTILES, 2 * _TBLK), np.int32)
    for w in range(_NTILES):
        rlo = _rlo(w)
        assert _W0 <= rlo and rlo + _RPT <= _W0 + _WROWS
        for s in range(2):
            base = s * _SIDE
            gate = base + (_BROWS - 1) * _H
            for r in range(_RPT):
                sy = idx[w * _RPT + r]
                for c in range(_OUT):
                    sx = idx[c]
                    p = s * _TBLK + r * _OUT + c
                    if sy >= 0 and sx >= 0:
                        lrow = sy - rlo
                        assert 0 <= lrow < _RPT, (w, r, sy, rlo)
                        tab[w, p] = base + lrow * _H + sx
                    else:
                        tab[w, p] = gate
    return tab.reshape(-1)


_R = _bilinear_mat()
_CFLAT = _flat_index_table()


def _tc_body(il_ref, ir_ref, r_ref, oil_ref, oir_ref):
    r = r_ref[...]
    rt = r_ref[...].T
    for x_ref, o_ref in ((il_ref, oil_ref), (ir_ref, oir_ref)):
        for p in range(3):
            t = jnp.dot(r, x_ref[p], preferred_element_type=jnp.float32)
            o_ref[p] = jnp.dot(t, rt, preferred_element_type=jnp.float32)


def _sc_flow(dl_hbm, dr_hbm, mfl_hbm, mfr_hbm, cflat_hbm,
             odl, odr, oml, omr,
             cidx_v, b0_v, b1_v, bm_v, o0_v, o1_v, om_v, sem):
    wid = lax.axis_index("s") * 2 + lax.axis_index("c")
    rlo = (4 * (_LO + _RPT * wid)) // 5 - 1 - _W0
    boff = pl.multiple_of(rlo * _H, 8)
    coff = pl.multiple_of(wid * (2 * _TBLK), 2 * _TBLK)
    idx_cp = pltpu.make_async_copy(
        cflat_hbm.at[pl.ds(coff, 2 * _TBLK)], cidx_v, sem)
    idx_cp.start()

    zero16 = jnp.zeros((16,), jnp.float32)
    gate0 = (_BROWS - 1) * _H

    @plsc.parallel_loop(0, _H // 16, unroll=2)
    def zbody(k):
        for buf in (b0_v, b1_v, bm_v):
            buf[pl.ds(gate0 + k * 16, 16)] = zero16
            buf[pl.ds(_SIDE + gate0 + k * 16, 16)] = zero16

    n = _RPT * _H
    copies = []
    for s, (d_hbm, mf_hbm) in enumerate(((dl_hbm, mfl_hbm), (dr_hbm, mfr_hbm))):
        sb = s * _SIDE
        copies += [
            pltpu.make_async_copy(d_hbm.at[pl.ds(boff, n)],
                                  b0_v.at[pl.ds(sb, n)], sem),
            pltpu.make_async_copy(d_hbm.at[pl.ds(_WROWS * _H + boff, n)],
                                  b1_v.at[pl.ds(sb, n)], sem),
            pltpu.make_async_copy(mf_hbm.at[pl.ds(boff, n)],
                                  bm_v.at[pl.ds(sb, n)], sem),
        ]
    for c in copies:
        c.start()
    with jax.named_scope("drain"):
        idx_cp.wait()
        for c in copies:
            c.wait()

    with jax.named_scope("colgather"):
        @plsc.parallel_loop(0, _NPOS, unroll=2)
        def jbody(i):
            pos = i * 16
            idx16 = cidx_v[pl.ds(pos, 16)]
            mg = plsc.load_gather(bm_v, [idx16])
            d0 = plsc.load_gather(b0_v, [idx16])
            d1 = plsc.load_gather(b1_v, [idx16])
            om_v[pl.ds(pos, 16)] = mg
            sm = mg * jnp.float32(_SCALE)
            o0_v[pl.ds(pos, 16)] = d0 * sm
            o1_v[pl.ds(pos, 16)] = d1 * sm

    with jax.named_scope("writeout"):
        ooff = pl.multiple_of(wid * _TBLK, _TBLK)
        pltpu.sync_copy(o0_v.at[pl.ds(0, _TBLK)], odl.at[pl.ds(ooff, _TBLK)])
        pltpu.sync_copy(o1_v.at[pl.ds(0, _TBLK)],
                        odl.at[pl.ds(_PLANE + ooff, _TBLK)])
        pltpu.sync_copy(om_v.at[pl.ds(0, _TBLK)], oml.at[pl.ds(ooff, _TBLK)])
        pltpu.sync_copy(o0_v.at[pl.ds(_TBLK, _TBLK)],
                        odr.at[pl.ds(ooff, _TBLK)])
        pltpu.sync_copy(o1_v.at[pl.ds(_TBLK, _TBLK)],
                        odr.at[pl.ds(_PLANE + ooff, _TBLK)])
        pltpu.sync_copy(om_v.at[pl.ds(_TBLK, _TBLK)], omr.at[pl.ds(ooff, _TBLK)])


_sc_call = functools.partial(
    pl.kernel,
    mesh=plsc.VectorSubcoreMesh(core_axis_name="c", subcore_axis_name="s"),
    compiler_params=pltpu.CompilerParams(
        use_tc_tiling_on_sc=False, needs_layout_passes=False),
    out_type=(
        jax.ShapeDtypeStruct((2 * _PLANE,), jnp.float32),
        jax.ShapeDtypeStruct((2 * _PLANE,), jnp.float32),
        jax.ShapeDtypeStruct((_PLANE,), jnp.float32),
        jax.ShapeDtypeStruct((_PLANE,), jnp.float32),
    ),
    scratch_types=[
        pltpu.VMEM((2 * _TBLK,), jnp.int32),
        pltpu.VMEM((2 * _SIDE,), jnp.float32),
        pltpu.VMEM((2 * _SIDE,), jnp.float32),
        pltpu.VMEM((2 * _SIDE,), jnp.float32),
        pltpu.VMEM((2 * _TBLK,), jnp.float32),
        pltpu.VMEM((2 * _TBLK,), jnp.float32),
        pltpu.VMEM((2 * _TBLK,), jnp.float32),
        pltpu.SemaphoreType.DMA,
    ],
)(_sc_flow)


def kernel(img_left, img_right, dsp_left, dsp_right, mask_left, mask_right):
    oil, oir = pl.pallas_call(
        _tc_body,
        out_shape=(
            jax.ShapeDtypeStruct((3, _OUT, _OUT), jnp.float32),
            jax.ShapeDtypeStruct((3, _OUT, _OUT), jnp.float32),
        ),
    )(img_left, img_right, jnp.asarray(_R))

    win = slice(_W0, _W0 + _WROWS)
    odl, odr, oml, omr = _sc_call(
        dsp_left[:, win, :].reshape(-1), dsp_right[:, win, :].reshape(-1),
        mask_left[win].astype(jnp.float32).reshape(-1),
        mask_right[win].astype(jnp.float32).reshape(-1),
        jnp.asarray(_CFLAT),
    )

    return (oil, oir,
            odl.reshape(2, _OUT, _OUT), odr.reshape(2, _OUT, _OUT),
            oml.reshape(_OUT, _OUT), omr.reshape(_OUT, _OUT))


# speedup vs baseline: 1.7589x; 1.0009x over previous
import functools

import numpy as np
import jax
import jax.numpy as jnp
from jax import lax
from jax.experimental import pallas as pl
from jax.experimental.pallas import tpu as pltpu
from jax.experimental.pallas import tpu_sc as plsc

_H = 512
_OUT = 384
_W0 = 96
_WROWS = 320
_LO = 128
_SCALE = 1.25
_NTILES = 32
_RPT = _OUT // _NTILES
_BROWS = 16
_SIDE = _BROWS * _H
_TBLK = _RPT * _OUT
_PLANE = _OUT * _OUT
_NPOS = 2 * _TBLK // 16


def _bilinear_mat():
    inv = _H / (_H * _SCALE)
    o = np.arange(_LO, _LO + _OUT, dtype=np.float64)
    s = (o + 0.5) * inv - 0.5
    k = np.arange(_H, dtype=np.float64)
    w = np.maximum(0.0, 1.0 - np.abs(s[:, None] - k[None, :]))
    w = w / w.sum(1, keepdims=True)
    return w.astype(np.float32)


def _gather_maps():
    src = np.arange(_H)
    tgt = np.round(src.astype(np.float32) * np.float32(_SCALE)).astype(np.int64)
    r = tgt - _LO
    ok = (r >= 0) & (r < _OUT)
    idx = np.full((_OUT,), -1, np.int64)
    idx[r[ok]] = src[ok]
    return idx


def _rlo(w):
    return (4 * (_LO + _RPT * w)) // 5 - 1


def _flat_index_table():
    idx = _gather_maps()
    tab = np.zeros((_NTILES, 2 * _TBLK), np.int32)
    for w in range(_NTILES):
        rlo = _rlo(w)
        assert _W0 <= rlo and rlo + _RPT <= _W0 + _WROWS
        for s in range(2):
            base = s * _SIDE
            gate = base + (_BROWS - 1) * _H
            for r in range(_RPT):
                sy = idx[w * _RPT + r]
                for c in range(_OUT):
                    sx = idx[c]
                    p = s * _TBLK + r * _OUT + c
                    if sy >= 0 and sx >= 0:
                        lrow = sy - rlo
                        assert 0 <= lrow < _RPT, (w, r, sy, rlo)
                        tab[w, p] = base + lrow * _H + sx
                    else:
                        tab[w, p] = gate
    return tab.reshape(-1)


_R = _bilinear_mat()
_CFLAT = _flat_index_table()


def _tc_body(il_ref, ir_ref, r_ref, oil_ref, oir_ref):
    r = r_ref[...]
    rt = r_ref[...].T
    for x_ref, o_ref in ((il_ref, oil_ref), (ir_ref, oir_ref)):
        for p in range(3):
            t = jnp.dot(r, x_ref[p], preferred_element_type=jnp.float32)
            o_ref[p] = jnp.dot(t, rt, preferred_element_type=jnp.float32)


def _sc_flow(dl_hbm, dr_hbm, mfl_hbm, mfr_hbm, cflat_hbm,
             odl, odr, oml, omr,
             cidx_v, b0_v, b1_v, bm_v, o0_v, o1_v, om_v, sem):
    wid = lax.axis_index("s") * 2 + lax.axis_index("c")
    rlo = (4 * (_LO + _RPT * wid)) // 5 - 1 - _W0
    boff = pl.multiple_of(rlo * _H, 8)
    coff = pl.multiple_of(wid * (2 * _TBLK), 2 * _TBLK)
    idx_cp = pltpu.make_async_copy(
        cflat_hbm.at[pl.ds(coff, 2 * _TBLK)], cidx_v, sem)
    idx_cp.start()

    zero16 = jnp.zeros((16,), jnp.float32)
    gate0 = (_BROWS - 1) * _H

    @plsc.parallel_loop(0, _H // 16, unroll=2)
    def zbody(k):
        for buf in (b0_v, b1_v, bm_v):
            buf[pl.ds(gate0 + k * 16, 16)] = zero16
            buf[pl.ds(_SIDE + gate0 + k * 16, 16)] = zero16

    n = _RPT * _H
    copies = []
    for s, (d_hbm, mf_hbm) in enumerate(((dl_hbm, mfl_hbm), (dr_hbm, mfr_hbm))):
        sb = s * _SIDE
        copies += [
            pltpu.make_async_copy(d_hbm.at[pl.ds(boff, n)],
                                  b0_v.at[pl.ds(sb, n)], sem),
            pltpu.make_async_copy(d_hbm.at[pl.ds(_WROWS * _H + boff, n)],
                                  b1_v.at[pl.ds(sb, n)], sem),
            pltpu.make_async_copy(mf_hbm.at[pl.ds(boff, n)],
                                  bm_v.at[pl.ds(sb, n)], sem),
        ]
    for c in copies:
        c.start()
    idx_cp.wait()
    for c in copies:
        c.wait()

    @plsc.parallel_loop(0, _NPOS, unroll=2)
    def jbody(i):
        pos = i * 16
        idx16 = cidx_v[pl.ds(pos, 16)]
        mg = plsc.load_gather(bm_v, [idx16])
        d0 = plsc.load_gather(b0_v, [idx16])
        d1 = plsc.load_gather(b1_v, [idx16])
        om_v[pl.ds(pos, 16)] = mg
        sm = mg * jnp.float32(_SCALE)
        o0_v[pl.ds(pos, 16)] = d0 * sm
        o1_v[pl.ds(pos, 16)] = d1 * sm

    ooff = pl.multiple_of(wid * _TBLK, _TBLK)
    pltpu.sync_copy(o0_v.at[pl.ds(0, _TBLK)], odl.at[pl.ds(ooff, _TBLK)])
    pltpu.sync_copy(o1_v.at[pl.ds(0, _TBLK)],
                    odl.at[pl.ds(_PLANE + ooff, _TBLK)])
    pltpu.sync_copy(om_v.at[pl.ds(0, _TBLK)], oml.at[pl.ds(ooff, _TBLK)])
    pltpu.sync_copy(o0_v.at[pl.ds(_TBLK, _TBLK)],
                    odr.at[pl.ds(ooff, _TBLK)])
    pltpu.sync_copy(o1_v.at[pl.ds(_TBLK, _TBLK)],
                    odr.at[pl.ds(_PLANE + ooff, _TBLK)])
    pltpu.sync_copy(om_v.at[pl.ds(_TBLK, _TBLK)], omr.at[pl.ds(ooff, _TBLK)])


_sc_call = functools.partial(
    pl.kernel,
    mesh=plsc.VectorSubcoreMesh(core_axis_name="c", subcore_axis_name="s"),
    compiler_params=pltpu.CompilerParams(
        use_tc_tiling_on_sc=False, needs_layout_passes=False),
    out_type=(
        jax.ShapeDtypeStruct((2 * _PLANE,), jnp.float32),
        jax.ShapeDtypeStruct((2 * _PLANE,), jnp.float32),
        jax.ShapeDtypeStruct((_PLANE,), jnp.float32),
        jax.ShapeDtypeStruct((_PLANE,), jnp.float32),
    ),
    scratch_types=[
        pltpu.VMEM((2 * _TBLK,), jnp.int32),
        pltpu.VMEM((2 * _SIDE,), jnp.float32),
        pltpu.VMEM((2 * _SIDE,), jnp.float32),
        pltpu.VMEM((2 * _SIDE,), jnp.float32),
        pltpu.VMEM((2 * _TBLK,), jnp.float32),
        pltpu.VMEM((2 * _TBLK,), jnp.float32),
        pltpu.VMEM((2 * _TBLK,), jnp.float32),
        pltpu.SemaphoreType.DMA,
    ],
)(_sc_flow)


def kernel(img_left, img_right, dsp_left, dsp_right, mask_left, mask_right):
    oil, oir = pl.pallas_call(
        _tc_body,
        out_shape=(
            jax.ShapeDtypeStruct((3, _OUT, _OUT), jnp.float32),
            jax.ShapeDtypeStruct((3, _OUT, _OUT), jnp.float32),
        ),
    )(img_left, img_right, jnp.asarray(_R))

    win = slice(_W0, _W0 + _WROWS)
    odl, odr, oml, omr = _sc_call(
        dsp_left[:, win, :].reshape(-1), dsp_right[:, win, :].reshape(-1),
        mask_left[win].astype(jnp.float32).reshape(-1),
        mask_right[win].astype(jnp.float32).reshape(-1),
        jnp.asarray(_CFLAT),
    )

    return (oil, oir,
            odl.reshape(2, _OUT, _OUT), odr.reshape(2, _OUT, _OUT),
            oml.reshape(_OUT, _OUT), omr.reshape(_OUT, _OUT))


# speedup vs baseline: 1.7647x; 1.0033x over previous
import functools

import numpy as np
import jax
import jax.numpy as jnp
from jax import lax
from jax.experimental import pallas as pl
from jax.experimental.pallas import tpu as pltpu
from jax.experimental.pallas import tpu_sc as plsc

_H = 512
_OUT = 384
_W0 = 96
_WROWS = 320
_LO = 128
_SCALE = 1.25
_NTILES = 32
_RPT = _OUT // _NTILES
_BROWS = 16
_SIDE = _BROWS * _H
_TBLK = _RPT * _OUT
_PLANE = _OUT * _OUT
_NPOS = 2 * _TBLK // 16


def _bilinear_mat():
    inv = _H / (_H * _SCALE)
    o = np.arange(_LO, _LO + _OUT, dtype=np.float64)
    s = (o + 0.5) * inv - 0.5
    k = np.arange(_H, dtype=np.float64)
    w = np.maximum(0.0, 1.0 - np.abs(s[:, None] - k[None, :]))
    w = w / w.sum(1, keepdims=True)
    return w.astype(np.float32)


def _gather_maps():
    src = np.arange(_H)
    tgt = np.round(src.astype(np.float32) * np.float32(_SCALE)).astype(np.int64)
    r = tgt - _LO
    ok = (r >= 0) & (r < _OUT)
    idx = np.full((_OUT,), -1, np.int64)
    idx[r[ok]] = src[ok]
    return idx


def _rlo(w):
    return (4 * (_LO + _RPT * w)) // 5 - 1


def _flat_index_table():
    idx = _gather_maps()
    tab = np.zeros((_NTILES, 2 * _TBLK), np.int32)
    for w in range(_NTILES):
        rlo = _rlo(w)
        assert _W0 <= rlo and rlo + _RPT <= _W0 + _WROWS
        for s in range(2):
            base = s * _SIDE
            gate = base + (_BROWS - 1) * _H
            for r in range(_RPT):
                sy = idx[w * _RPT + r]
                for c in range(_OUT):
                    sx = idx[c]
                    p = s * _TBLK + r * _OUT + c
                    if sy >= 0 and sx >= 0:
                        lrow = sy - rlo
                        assert 0 <= lrow < _RPT, (w, r, sy, rlo)
                        tab[w, p] = base + lrow * _H + sx
                    else:
                        tab[w, p] = gate
    return tab.reshape(-1)


_R = _bilinear_mat()
_CFLAT = _flat_index_table()


def _tc_body(il_ref, ir_ref, r_ref, oil_ref, oir_ref):
    r = r_ref[...]
    rt = r_ref[...].T
    for x_ref, o_ref in ((il_ref, oil_ref), (ir_ref, oir_ref)):
        for p in range(3):
            t = jnp.dot(r, x_ref[p], preferred_element_type=jnp.float32)
            o_ref[p] = jnp.dot(t, rt, preferred_element_type=jnp.float32)


def _sc_flow(dl_hbm, dr_hbm, mfl_hbm, mfr_hbm, cflat_hbm,
             odl, odr, oml, omr,
             cidx_v, b0_v, b1_v, bm_v, o0_v, o1_v, om_v, sem):
    wid = lax.axis_index("s") * 2 + lax.axis_index("c")
    rlo = (4 * (_LO + _RPT * wid)) // 5 - 1 - _W0
    boff = pl.multiple_of(rlo * _H, 8)
    coff = pl.multiple_of(wid * (2 * _TBLK), 2 * _TBLK)
    idx_cp = pltpu.make_async_copy(
        cflat_hbm.at[pl.ds(coff, 2 * _TBLK)], cidx_v, sem)
    idx_cp.start()

    n = _RPT * _H
    copies = []
    for s, (d_hbm, mf_hbm) in enumerate(((dl_hbm, mfl_hbm), (dr_hbm, mfr_hbm))):
        sb = s * _SIDE
        copies += [
            pltpu.make_async_copy(d_hbm.at[pl.ds(boff, n)],
                                  b0_v.at[pl.ds(sb, n)], sem),
            pltpu.make_async_copy(d_hbm.at[pl.ds(_WROWS * _H + boff, n)],
                                  b1_v.at[pl.ds(sb, n)], sem),
            pltpu.make_async_copy(mf_hbm.at[pl.ds(boff, n)],
                                  bm_v.at[pl.ds(sb, n)], sem),
        ]
    for c in copies:
        c.start()

    zero16 = jnp.zeros((16,), jnp.float32)
    gate0 = (_BROWS - 1) * _H

    @plsc.parallel_loop(0, _H // 16, unroll=2)
    def zbody(k):
        for buf in (b0_v, b1_v, bm_v):
            buf[pl.ds(gate0 + k * 16, 16)] = zero16
            buf[pl.ds(_SIDE + gate0 + k * 16, 16)] = zero16

    idx_cp.wait()
    for c in copies:
        c.wait()

    @plsc.parallel_loop(0, _NPOS, unroll=4)
    def jbody(i):
        pos = i * 16
        idx16 = cidx_v[pl.ds(pos, 16)]
        mg = plsc.load_gather(bm_v, [idx16])
        d0 = plsc.load_gather(b0_v, [idx16])
        d1 = plsc.load_gather(b1_v, [idx16])
        om_v[pl.ds(pos, 16)] = mg
        sm = mg * jnp.float32(_SCALE)
        o0_v[pl.ds(pos, 16)] = d0 * sm
        o1_v[pl.ds(pos, 16)] = d1 * sm

    ooff = pl.multiple_of(wid * _TBLK, _TBLK)
    pltpu.sync_copy(o0_v.at[pl.ds(0, _TBLK)], odl.at[pl.ds(ooff, _TBLK)])
    pltpu.sync_copy(o1_v.at[pl.ds(0, _TBLK)],
                    odl.at[pl.ds(_PLANE + ooff, _TBLK)])
    pltpu.sync_copy(om_v.at[pl.ds(0, _TBLK)], oml.at[pl.ds(ooff, _TBLK)])
    pltpu.sync_copy(o0_v.at[pl.ds(_TBLK, _TBLK)],
                    odr.at[pl.ds(ooff, _TBLK)])
    pltpu.sync_copy(o1_v.at[pl.ds(_TBLK, _TBLK)],
                    odr.at[pl.ds(_PLANE + ooff, _TBLK)])
    pltpu.sync_copy(om_v.at[pl.ds(_TBLK, _TBLK)], omr.at[pl.ds(ooff, _TBLK)])


_sc_call = functools.partial(
    pl.kernel,
    mesh=plsc.VectorSubcoreMesh(core_axis_name="c", subcore_axis_name="s"),
    compiler_params=pltpu.CompilerParams(
        use_tc_tiling_on_sc=False, needs_layout_passes=False),
    out_type=(
        jax.ShapeDtypeStruct((2 * _PLANE,), jnp.float32),
        jax.ShapeDtypeStruct((2 * _PLANE,), jnp.float32),
        jax.ShapeDtypeStruct((_PLANE,), jnp.float32),
        jax.ShapeDtypeStruct((_PLANE,), jnp.float32),
    ),
    scratch_types=[
        pltpu.VMEM((2 * _TBLK,), jnp.int32),
        pltpu.VMEM((2 * _SIDE,), jnp.float32),
        pltpu.VMEM((2 * _SIDE,), jnp.float32),
        pltpu.VMEM((2 * _SIDE,), jnp.float32),
        pltpu.VMEM((2 * _TBLK,), jnp.float32),
        pltpu.VMEM((2 * _TBLK,), jnp.float32),
        pltpu.VMEM((2 * _TBLK,), jnp.float32),
        pltpu.SemaphoreType.DMA,
    ],
)(_sc_flow)


def kernel(img_left, img_right, dsp_left, dsp_right, mask_left, mask_right):
    oil, oir = pl.pallas_call(
        _tc_body,
        out_shape=(
            jax.ShapeDtypeStruct((3, _OUT, _OUT), jnp.float32),
            jax.ShapeDtypeStruct((3, _OUT, _OUT), jnp.float32),
        ),
    )(img_left, img_right, jnp.asarray(_R))

    win = slice(_W0, _W0 + _WROWS)
    odl, odr, oml, omr = _sc_call(
        dsp_left[:, win, :].reshape(-1), dsp_right[:, win, :].reshape(-1),
        mask_left[win].astype(jnp.float32).reshape(-1),
        mask_right[win].astype(jnp.float32).reshape(-1),
        jnp.asarray(_CFLAT),
    )

    return (oil, oir,
            odl.reshape(2, _OUT, _OUT), odr.reshape(2, _OUT, _OUT),
            oml.reshape(_OUT, _OUT), omr.reshape(_OUT, _OUT))
